# Initial kernel scaffold; baseline (speedup 1.0000x reference)
import jax
import jax.numpy as jnp
from jax.experimental import pallas as pl


def kernel(x, edge_index, focal_seq, action_child, action_parent, action_time, W_embed, b_embed, Wl_bu, bl_bu, Wr_bu, br_bu, att_bu, bias_bu, Wl_td, bl_td, Wr_td, br_td, att_td, bias_td, time_table, W_comb, b_comb, W_mlp1, b_mlp1, W_mlp2, b_mlp2, W_seq, b_seq):
    raise NotImplementedError("write your pallas kernel here")


# speedup vs baseline: 6.4483x; 6.4483x over previous
import functools

import jax
import jax.numpy as jnp
from jax import lax
from jax.experimental import pallas as pl
from jax.experimental.pallas import tpu as pltpu
from jax.experimental.pallas import tpu_sc as plsc

N = 10000
E = 320000
D = 128
DH = D // 2
NC = 2
NS = 16
NW1 = NC * NS
EPW1 = E // NW1
EPW2 = E // NS
CH = 80
NCH1 = EPW1 // CH
NCH2 = EPW2 // CH
ZB = 80
NZCH = N // ZB
NZIT = -(-NZCH // NS)
NA = 1024
APW = NA // (NC * NS)

_f32 = jnp.float32


def _mesh():
    return plsc.VectorSubcoreMesh(core_axis_name="c", subcore_axis_name="s")


def _alpha_body(xl_h, xr_h, src_h, dst_h, att_h, wexp_h, den_h,
                idx_s, idx_d, rows_l, rows_r, wden, att_v, zb_d,
                acc_d, sem_l, sem_r):
    cid = lax.axis_index("c")
    sid = lax.axis_index("s")
    wid = sid * NC + cid

    z16 = jnp.zeros((16,), _f32)

    def _zrow(r, c):
        zb_d[r, :] = z16
        return c
    lax.fori_loop(0, ZB, _zrow, 0)

    def _zcp(j, c):
        cj = sid + NS * j

        @pl.when(cj < NZCH)
        def _():
            pltpu.sync_copy(zb_d, acc_d.at[pl.ds(cj * ZB, ZB)])
        return c
    lax.fori_loop(0, NZIT, _zcp, 0)

    pltpu.sync_copy(att_h, att_v)
    pltpu.sync_copy(src_h.at[wid], idx_s)
    pltpu.sync_copy(dst_h.at[wid], idx_d)
    plsc.subcore_barrier()

    att_k = [att_v[pl.ds(16 * k, 16)] for k in range(8)]

    perms = [jnp.bitwise_xor(lax.iota(jnp.int32, 16), s) for s in (8, 4, 2, 1)]
    gd = lax.GatherDimensionNumbers(
        offset_dims=(), collapsed_slice_dims=(0,), start_index_map=(0,))

    def _lane_sum(v):
        for p in perms:
            v = v + lax.gather(v, p[:, None], gd, (1,),
                               mode=lax.GatherScatterMode.PROMISE_IN_BOUNDS)
        return v

    def _chunk(j, c):
        gl = pltpu.async_copy(xl_h.at[idx_s.at[j]], rows_l, sem_l)
        gr = pltpu.async_copy(xr_h.at[idx_d.at[j]], rows_r, sem_r)
        gl.wait()
        gr.wait()

        def _edge(i, cc):
            acc = None
            for k in range(8):
                e = rows_l[i, pl.ds(16 * k, 16)] + rows_r[i, pl.ds(16 * k, 16)]
                e = jnp.where(e > 0, e, 0.2 * e)
                t = e * att_k[k]
                acc = t if acc is None else acc + t
            wden[i, :] = jnp.exp(_lane_sum(acc))
            return cc
        lax.fori_loop(0, CH, _edge, 0)

        pltpu.sync_copy(wden, wexp_h.at[pl.ds(wid * EPW1 + j * CH, CH)])
        pltpu.sync_copy(wden, acc_d.at[idx_d.at[j]], add=True)
        return c
    lax.fori_loop(0, NCH1, _chunk, 0)

    plsc.subcore_barrier()

    def _flush(j, c):
        cj = sid + NS * j

        @pl.when(cj < NZCH)
        def _():
            pltpu.sync_copy(acc_d.at[pl.ds(cj * ZB, ZB)],
                            den_h.at[cid, pl.ds(cj * ZB, ZB)])
        return c
    lax.fori_loop(0, NZIT, _flush, 0)


def _sc_alpha(xl, xr, src3d, dst3d, att):
    fn = functools.partial(
        pl.kernel, mesh=_mesh(),
        compiler_params=pltpu.CompilerParams(use_tc_tiling_on_sc=False),
        out_type=[jax.ShapeDtypeStruct((E, 16), _f32),
                  jax.ShapeDtypeStruct((NC, N, 16), _f32)],
        scratch_types=[
            pltpu.VMEM((NCH1, CH), jnp.int32),
            pltpu.VMEM((NCH1, CH), jnp.int32),
            pltpu.VMEM((CH, D), _f32),
            pltpu.VMEM((CH, D), _f32),
            pltpu.VMEM((CH, 16), _f32),
            pltpu.VMEM((D,), _f32),
            pltpu.VMEM((ZB, 16), _f32),
            pltpu.VMEM_SHARED((N, 16), _f32),
            pltpu.SemaphoreType.DMA,
            pltpu.SemaphoreType.DMA,
        ],
    )(_alpha_body)
    return fn(xl, xr, src3d, dst3d, att)


def _scat_body(xh_h, wexp_h, src_h, dst_h, numh_h,
               idx_s, idx_d, rows_h, wbuf, wnum, zb_n,
               acc_h, sem_g):
    cid = lax.axis_index("c")
    sid = lax.axis_index("s")

    z16 = jnp.zeros((16,), _f32)

    def _zrow(r, c):
        for k in range(DH // 16):
            zb_n[r, pl.ds(16 * k, 16)] = z16
        return c
    lax.fori_loop(0, ZB, _zrow, 0)

    def _zcp(j, c):
        cj = sid + NS * j

        @pl.when(cj < NZCH)
        def _():
            pltpu.sync_copy(zb_n, acc_h.at[pl.ds(cj * ZB, ZB)])
        return c
    lax.fori_loop(0, NZIT, _zcp, 0)

    pltpu.sync_copy(src_h.at[sid], idx_s)
    pltpu.sync_copy(dst_h.at[sid], idx_d)
    plsc.subcore_barrier()

    xh = xh_h.at[cid]

    def _chunk(j, c):
        g = pltpu.async_copy(xh.at[idx_s.at[j]], rows_h, sem_g)
        pltpu.sync_copy(wexp_h.at[pl.ds(sid * EPW2 + j * CH, CH)], wbuf)
        g.wait()

        def _edge(i, cc):
            w = wbuf[i, :]
            for k in range(DH // 16):
                wnum[i, pl.ds(16 * k, 16)] = w * rows_h[i, pl.ds(16 * k, 16)]
            return cc
        lax.fori_loop(0, CH, _edge, 0)

        pltpu.sync_copy(wnum, acc_h.at[idx_d.at[j]], add=True)
        return c
    lax.fori_loop(0, NCH2, _chunk, 0)

    plsc.subcore_barrier()

    def _flush(j, c):
        cj = sid + NS * j

        @pl.when(cj < NZCH)
        def _():
            pltpu.sync_copy(acc_h.at[pl.ds(cj * ZB, ZB)],
                            numh_h.at[cid, pl.ds(cj * ZB, ZB)])
        return c
    lax.fori_loop(0, NZIT, _flush, 0)


def _sc_scatter(xh, wexp, src3d, dst3d):
    fn = functools.partial(
        pl.kernel, mesh=_mesh(),
        compiler_params=pltpu.CompilerParams(use_tc_tiling_on_sc=False),
        out_type=jax.ShapeDtypeStruct((NC, N, DH), _f32),
        scratch_types=[
            pltpu.VMEM((NCH2, CH), jnp.int32),
            pltpu.VMEM((NCH2, CH), jnp.int32),
            pltpu.VMEM((CH, DH), _f32),
            pltpu.VMEM((CH, 16), _f32),
            pltpu.VMEM((CH, DH), _f32),
            pltpu.VMEM((ZB, DH), _f32),
            pltpu.VMEM_SHARED((N, DH), _f32),
            pltpu.SemaphoreType.DMA,
        ],
    )(_scat_body)
    return fn(xh, wexp, src3d, dst3d)


def _gat_edge_pass(xl, xlh, xr, src3d_1, dst3d_1, src3d_2, dst3d_2, att):
    wexp, den = _sc_alpha(xl, xr, src3d_1, dst3d_1, att)
    numh = _sc_scatter(xlh, wexp, src3d_2, dst3d_2)
    return numh, den


def _gather_body(h3, tt, ci, pi, ti, hc, hp, te,
                 i1, i2, i3, r1, r2, r3, s1, s2, s3):
    cid = lax.axis_index("c")
    sid = lax.axis_index("s")
    b = (sid * NC + cid) * APW
    pltpu.sync_copy(ci.at[pl.ds(b, APW)], i1)
    pltpu.sync_copy(pi.at[pl.ds(b, APW)], i2)
    pltpu.sync_copy(ti.at[pl.ds(b, APW)], i3)
    c1 = pltpu.async_copy(h3.at[i1], r1, s1)
    c2 = pltpu.async_copy(h3.at[i2], r2, s2)
    c3 = pltpu.async_copy(tt.at[i3], r3, s3)
    c1.wait()
    c2.wait()
    c3.wait()
    pltpu.sync_copy(r1, hc.at[pl.ds(b, APW)])
    pltpu.sync_copy(r2, hp.at[pl.ds(b, APW)])
    pltpu.sync_copy(r3, te.at[pl.ds(b, APW)])


def _sc_gather(h3, tt, ci, pi, ti):
    fn = functools.partial(
        pl.kernel, mesh=_mesh(),
        compiler_params=pltpu.CompilerParams(use_tc_tiling_on_sc=False),
        out_type=[jax.ShapeDtypeStruct((NA, D), _f32)] * 3,
        scratch_types=[
            pltpu.VMEM((APW,), jnp.int32),
            pltpu.VMEM((APW,), jnp.int32),
            pltpu.VMEM((APW,), jnp.int32),
            pltpu.VMEM((APW, D), _f32),
            pltpu.VMEM((APW, D), _f32),
            pltpu.VMEM((APW, D), _f32),
            pltpu.SemaphoreType.DMA,
            pltpu.SemaphoreType.DMA,
            pltpu.SemaphoreType.DMA,
        ],
    )(_gather_body)
    return fn(h3, tt, ci, pi, ti)


TB = 1000


def _split_halves(x_block):
    return x_block[:, :DH], x_block[:, DH:]


def _tc1_body(x_ref, wet, bet, wlt, blt, wrt, brt,
              h_ref, xl_ref, xlh_ref, xr_ref):
    h = jnp.dot(x_ref[...], wet[...], preferred_element_type=_f32) + bet[...]
    h_ref[...] = h
    xl = jnp.dot(h, wlt[...], preferred_element_type=_f32) + blt[...]
    xl_ref[...] = xl
    a, b = _split_halves(xl)
    xlh_ref[0] = a
    xlh_ref[1] = b
    xr_ref[...] = jnp.dot(h, wrt[...], preferred_element_type=_f32) + brt[...]


def _tc1(x, wet, bet, wlt, blt, wrt, brt):
    row = pl.BlockSpec((TB, D), lambda i: (i, 0))
    half = pl.BlockSpec((NC, TB, DH), lambda i: (0, i, 0))
    mat = pl.BlockSpec((D, D), lambda i: (0, 0))
    vec = pl.BlockSpec((1, D), lambda i: (0, 0))
    return pl.pallas_call(
        _tc1_body,
        grid=(N // TB,),
        in_specs=[row, mat, vec, mat, vec, mat, vec],
        out_specs=[row, row, half, row],
        out_shape=[jax.ShapeDtypeStruct((N, D), _f32),
                   jax.ShapeDtypeStruct((N, D), _f32),
                   jax.ShapeDtypeStruct((NC, N, DH), _f32),
                   jax.ShapeDtypeStruct((N, D), _f32)],
    )(x, wet, bet, wlt, blt, wrt, brt)


def _gat_out(numh_ref, den_ref, bias_ref):
    num = jnp.concatenate([numh_ref[0], numh_ref[1]], axis=-1)
    den = den_ref[0] + den_ref[1]
    return num / (den[:, 0:1] + 1e-16) + bias_ref[...]


def _tc2_body(numh_ref, den_ref, h_ref, bias_ref, wlt, blt, wrt, brt,
              h2_ref, xl_ref, xlh_ref, xr_ref):
    g = _gat_out(numh_ref, den_ref, bias_ref)
    h2 = h_ref[...] + jnp.maximum(g, 0.0)
    h2_ref[...] = h2
    xl = jnp.dot(h2, wlt[...], preferred_element_type=_f32) + blt[...]
    xl_ref[...] = xl
    a, b = _split_halves(xl)
    xlh_ref[0] = a
    xlh_ref[1] = b
    xr_ref[...] = jnp.dot(h2, wrt[...], preferred_element_type=_f32) + brt[...]


def _tc2(numh, den, h, bias, wlt, blt, wrt, brt):
    row = pl.BlockSpec((TB, D), lambda i: (i, 0))
    half = pl.BlockSpec((NC, TB, DH), lambda i: (0, i, 0))
    dspec = pl.BlockSpec((NC, TB, 16), lambda i: (0, i, 0))
    mat = pl.BlockSpec((D, D), lambda i: (0, 0))
    vec = pl.BlockSpec((1, D), lambda i: (0, 0))
    return pl.pallas_call(
        _tc2_body,
        grid=(N // TB,),
        in_specs=[half, dspec, row, vec, mat, vec, mat, vec],
        out_specs=[row, row, half, row],
        out_shape=[jax.ShapeDtypeStruct((N, D), _f32),
                   jax.ShapeDtypeStruct((N, D), _f32),
                   jax.ShapeDtypeStruct((NC, N, DH), _f32),
                   jax.ShapeDtypeStruct((N, D), _f32)],
    )(numh, den, h, bias, wlt, blt, wrt, brt)


def _tc3_body(numh_ref, den_ref, h_ref, bias_ref, h3_ref):
    g = _gat_out(numh_ref, den_ref, bias_ref)
    h3_ref[...] = h_ref[...] + jnp.maximum(g, 0.0)


def _tc3(numh, den, h, bias):
    row = pl.BlockSpec((TB, D), lambda i: (i, 0))
    half = pl.BlockSpec((NC, TB, DH), lambda i: (0, i, 0))
    dspec = pl.BlockSpec((NC, TB, 16), lambda i: (0, i, 0))
    vec = pl.BlockSpec((1, D), lambda i: (0, 0))
    return pl.pallas_call(
        _tc3_body,
        grid=(N // TB,),
        in_specs=[half, dspec, row, vec],
        out_specs=row,
        out_shape=jax.ShapeDtypeStruct((N, D), _f32),
    )(numh, den, h, bias)


def _tc4_body(hc_ref, hp_ref, te_ref, focal_ref, wseqt, bseq,
              wcc, wcp, bc, w1a, w1b, w1c, b1, w2, b2, out_ref):
    ctx = jnp.dot(focal_ref[...], wseqt[...], preferred_element_type=_f32) + bseq[...]
    branch = (jnp.dot(hc_ref[...], wcc[...], preferred_element_type=_f32)
              + jnp.dot(hp_ref[...], wcp[...], preferred_element_type=_f32)
              + bc[...])
    pre = (jnp.dot(branch, w1a[...], preferred_element_type=_f32)
           + jnp.dot(te_ref[...], w1b[...], preferred_element_type=_f32)
           + jnp.dot(ctx, w1c[...], preferred_element_type=_f32)
           + b1[...])
    hid = jnp.maximum(pre, 0.0)
    out_ref[...] = jnp.sum(hid * w2[...], axis=1, keepdims=True) + b2[...]


def _tc4(hc, hp, te, focal, wseqt, bseq, wcc, wcp, bc, w1a, w1b, w1c, b1, w2, b2):
    return pl.pallas_call(
        _tc4_body,
        out_shape=jax.ShapeDtypeStruct((NA, 1), _f32),
    )(hc, hp, te, focal, wseqt, bseq, wcc, wcp, bc, w1a, w1b, w1c, b1, w2, b2)


def kernel(x, edge_index, focal_seq, action_child, action_parent, action_time,
           W_embed, b_embed, Wl_bu, bl_bu, Wr_bu, br_bu, att_bu, bias_bu,
           Wl_td, bl_td, Wr_td, br_td, att_td, bias_td, time_table,
           W_comb, b_comb, W_mlp1, b_mlp1, W_mlp2, b_mlp2, W_seq, b_seq):
    ei = edge_index.astype(jnp.int32)
    e0_1 = ei[0].reshape(NW1, NCH1, CH)
    e1_1 = ei[1].reshape(NW1, NCH1, CH)
    e0_2 = ei[0].reshape(NS, NCH2, CH)
    e1_2 = ei[1].reshape(NS, NCH2, CH)

    h, xl1, xl1h, xr1 = _tc1(x, W_embed.T, b_embed.reshape(1, D),
                             Wl_bu.T, bl_bu.reshape(1, D),
                             Wr_bu.T, br_bu.reshape(1, D))
    numh1, den1 = _gat_edge_pass(xl1, xl1h, xr1, e1_1, e0_1, e1_2, e0_2, att_bu)
    h2, xl2, xl2h, xr2 = _tc2(numh1, den1, h, bias_bu.reshape(1, D),
                              Wl_td.T, bl_td.reshape(1, D),
                              Wr_td.T, br_td.reshape(1, D))
    numh2, den2 = _gat_edge_pass(xl2, xl2h, xr2, e0_1, e1_1, e0_2, e1_2, att_td)
    h3 = _tc3(numh2, den2, h2, bias_td.reshape(1, D))

    ci = action_child.astype(jnp.int32)
    pi = jnp.clip(action_parent, 0).astype(jnp.int32)
    ti = action_time.astype(jnp.int32)
    hc, hp, te = _sc_gather(h3, time_table, ci, pi, ti)

    logits = _tc4(hc, hp, te, focal_seq,
                  W_seq.T, b_seq.reshape(1, D),
                  W_comb[:, :D].T, W_comb[:, D:].T, b_comb.reshape(1, D),
                  W_mlp1[:, :D].T, W_mlp1[:, D:2 * D].T, W_mlp1[:, 2 * D:].T,
                  b_mlp1.reshape(1, D), W_mlp2, b_mlp2.reshape(1, 1))
    return logits
